# baseline (speedup 1.0000x reference)
import functools

import jax
import jax.numpy as jnp
from jax import lax
from jax.experimental import pallas as pl
from jax.experimental.pallas import tpu as pltpu
from jax.experimental.pallas import tpu_sc as plsc

B, L, D = 1024, 50, 64
NLP = 768
BL = B * L

NC, NS = 2, 16
NW = NC * NS
CB = 80
CPW = BL // (NW * CB)


def _sc_gather5(exe_t, skill_t, resp_t, i_exe, i_skill, i_r, o_exe, o_skill):
    mesh = plsc.VectorSubcoreMesh(core_axis_name="c", subcore_axis_name="s")

    @functools.partial(
        pl.kernel,
        mesh=mesh,
        out_type=[jax.ShapeDtypeStruct((BL, D), jnp.float32)] * 5,
        scratch_types=[
            pltpu.VMEM((CPW, CB), jnp.int32),
            pltpu.VMEM((CB, D), jnp.float32),
            pltpu.SemaphoreType.DMA,
        ],
        compiler_params=pltpu.CompilerParams(use_tc_tiling_on_sc=False),
    )
    def k(exe_hbm, skill_hbm, resp_hbm, iexe_hbm, iskill_hbm, ir_hbm,
          oexe_hbm, oskill_hbm,
          gexe, gskill, gresp, goexe, goskill,
          idx_v, rows_v, sem):
        wid = lax.axis_index("s") * NC + lax.axis_index("c")
        row0 = wid * CPW

        def gather_one(table, idx_hbm, out_hbm):
            pltpu.sync_copy(idx_hbm.at[wid], idx_v)

            def chunk(j, carry):
                pltpu.async_copy(table.at[idx_v.at[j]], rows_v, sem).wait()
                start = pl.multiple_of((row0 + j) * CB, 8)
                pltpu.sync_copy(rows_v, out_hbm.at[pl.ds(start, CB)])
                return carry

            lax.fori_loop(0, CPW, chunk, 0)

        gather_one(exe_hbm, iexe_hbm, gexe)
        gather_one(skill_hbm, iskill_hbm, gskill)
        gather_one(resp_hbm, ir_hbm, gresp)
        gather_one(exe_hbm, oexe_hbm, goexe)
        gather_one(skill_hbm, oskill_hbm, goskill)

    return k(exe_t, skill_t, resp_t, i_exe, i_skill, i_r, o_exe, o_skill)


_BB = 16


def _tc_body(x1_ref, x2_ref, el_ref, ge_ref, gs_ref, gr_ref, goe_ref,
             gos_ref, pos_ref, wn_ref, bn_ref, wt_ref, bt_ref,
             enc_ref, dec_ref, out_ref):
    w = wn_ref[...]
    pos = pos_ref[...][None, :, :]
    bn = bn_ref[...][None, :, :]

    x1 = x1_ref[...].reshape(_BB * L, NLP)
    y1 = jnp.dot(x1, w, preferred_element_type=jnp.float32)
    y1 = y1.reshape(_BB, L, D)
    enc_ref[...] = y1 + bn + pos + ge_ref[...] + gs_ref[...]

    el = el_ref[...]
    dec_ref[...] = (el * wt_ref[...][None, :, :] + bt_ref[...][None, :, :]
                    + pos + gr_ref[...])

    x2 = x2_ref[...].reshape(_BB * L, NLP)
    y2 = jnp.dot(x2, w, preferred_element_type=jnp.float32)
    y2 = y2.reshape(_BB, L, D)
    out_ref[...] = y2 + bn + goe_ref[...] + gos_ref[...]


def _tc_combine(x_in, x_out, elapsed, g_exe, g_skill, g_resp, g_oexe,
                g_oskill, pos, w_nlp, b_nlp, w_time, b_time):
    big = pl.BlockSpec((_BB, L, NLP), lambda i: (i, 0, 0))
    tok = pl.BlockSpec((_BB, L, D), lambda i: (i, 0, 0))
    return pl.pallas_call(
        _tc_body,
        grid=(B // _BB,),
        in_specs=[
            big, big,
            pl.BlockSpec((_BB, L, 1), lambda i: (i, 0, 0)),
            tok, tok, tok, tok, tok,
            pl.BlockSpec((L, D), lambda i: (0, 0)),
            pl.BlockSpec((NLP, D), lambda i: (0, 0)),
            pl.BlockSpec((1, D), lambda i: (0, 0)),
            pl.BlockSpec((1, D), lambda i: (0, 0)),
            pl.BlockSpec((1, D), lambda i: (0, 0)),
        ],
        out_specs=[tok, tok, tok],
        out_shape=[jax.ShapeDtypeStruct((B, L, D), jnp.float32)] * 3,
    )(x_in, x_out, elapsed, g_exe, g_skill, g_resp, g_oexe, g_oskill,
      pos, w_nlp, b_nlp, w_time, b_time)


def kernel(input_nlp_embedding, input_exercise, input_skill, input_r,
           in_elapsed_time, output_nlp_embedding, out_exercise, out_skill,
           exercise_table, skill_table, response_table, position_table,
           W_time, b_time, W_nlp, b_nlp):
    def idx2d(a):
        return a.astype(jnp.int32).reshape(NW, CPW, CB)

    g_exe, g_skill, g_resp, g_oexe, g_oskill = _sc_gather5(
        exercise_table, skill_table, response_table,
        idx2d(input_exercise), idx2d(input_skill), idx2d(input_r),
        idx2d(out_exercise), idx2d(out_skill))

    def tok3d(a):
        return a.reshape(B, L, D)

    enc, dec, out = _tc_combine(
        input_nlp_embedding, output_nlp_embedding, in_elapsed_time,
        tok3d(g_exe), tok3d(g_skill), tok3d(g_resp), tok3d(g_oexe),
        tok3d(g_oskill), position_table, W_nlp,
        b_nlp.reshape(1, D), W_time, b_time.reshape(1, D))
    return (enc, dec, out)


# speedup vs baseline: 1.9161x; 1.9161x over previous
import functools

import jax
import jax.numpy as jnp
from jax import lax
from jax.experimental import pallas as pl
from jax.experimental.pallas import tpu as pltpu
from jax.experimental.pallas import tpu_sc as plsc

B, L, D = 1024, 50, 64
NLP = 768
NR = 4
BL = B * L

NC, NS = 2, 16
NW = NC * NS
CB = 80
CPW = BL // (NW * CB)
NG = 4
NBUF = 4
LAG = 3


def _sc_gather4(exe_t, skill_t, idx_all):
    mesh = plsc.VectorSubcoreMesh(core_axis_name="c", subcore_axis_name="s")

    @functools.partial(
        pl.kernel,
        mesh=mesh,
        out_type=[jax.ShapeDtypeStruct((BL, D), jnp.float32)] * NG,
        scratch_types=[
            pltpu.VMEM((NG, CPW, CB), jnp.int32),
            *[pltpu.VMEM((CB, D), jnp.float32) for _ in range(NBUF)],
            *[pltpu.SemaphoreType.DMA for _ in range(2 * NBUF)],
        ],
        compiler_params=pltpu.CompilerParams(use_tc_tiling_on_sc=False),
    )
    def k(exe_hbm, skill_hbm, idx_hbm,
          gexe, gskill, goexe, goskill,
          idx_v, *bufs_and_sems):
        bufs = bufs_and_sems[:NBUF]
        gsem = bufs_and_sems[NBUF:2 * NBUF]
        wsem = bufs_and_sems[2 * NBUF:]
        wid = lax.axis_index("s") * NC + lax.axis_index("c")
        base = wid * (CPW * CB)

        pltpu.sync_copy(idx_hbm.at[wid], idx_v)

        tables = [exe_hbm, skill_hbm, exe_hbm, skill_hbm]
        outs = [gexe, gskill, goexe, goskill]
        tasks = [(g, j) for j in range(CPW) for g in range(NG)]
        nt = len(tasks)
        ghandle = [None] * nt
        whandle = [None] * nt

        def drain(t):
            g, j = tasks[t]
            s = t % NBUF
            ghandle[t].wait()
            start = pl.multiple_of(base + j * CB, 8)
            whandle[t] = pltpu.async_copy(
                bufs[s], outs[g].at[pl.ds(start, CB)], wsem[s])

        for i in range(nt):
            s = i % NBUF
            if i >= NBUF:
                whandle[i - NBUF].wait()
            g, j = tasks[i]
            ghandle[i] = pltpu.async_copy(
                tables[g].at[idx_v.at[g, j]], bufs[s], gsem[s])
            if i >= LAG:
                drain(i - LAG)
        for t in range(nt - LAG, nt):
            drain(t)
        for t in range(nt - NBUF, nt):
            whandle[t].wait()

    return k(exe_t, skill_t, idx_all)


def _tc_body(x1_ref, x2_ref, el_ref, r_ref, ge_ref, gs_ref, goe_ref,
             gos_ref, pos_ref, wn_ref, bn_ref, wt_ref, bt_ref, resp_ref,
             enc_ref, dec_ref, out_ref):
    w = wn_ref[...]
    pos_l = pos_ref[...].reshape(1, D)
    bn = bn_ref[...]

    x1 = x1_ref[...].reshape(B, NLP)
    y1 = jnp.dot(x1, w, preferred_element_type=jnp.float32)
    enc = (y1 + bn + pos_l + ge_ref[...].reshape(B, D)
           + gs_ref[...].reshape(B, D))
    enc_ref[...] = enc.reshape(1, B, D)

    el = el_ref[...].reshape(B, 1)
    r = r_ref[...].reshape(B, 1)
    onehot = (r == lax.broadcasted_iota(jnp.int32, (1, NR), 1)
              ).astype(jnp.float32)
    dec_r = jnp.dot(onehot, resp_ref[...], preferred_element_type=jnp.float32)
    dec = el * wt_ref[...] + bt_ref[...] + pos_l + dec_r
    dec_ref[...] = dec.reshape(1, B, D)

    x2 = x2_ref[...].reshape(B, NLP)
    y2 = jnp.dot(x2, w, preferred_element_type=jnp.float32)
    out = (y2 + bn + goe_ref[...].reshape(B, D)
           + gos_ref[...].reshape(B, D))
    out_ref[...] = out.reshape(1, B, D)


def _tc_combine(x1_t, x2_t, el_t, r_t, g_exe, g_skill, g_oexe, g_oskill,
                pos, w_nlp, b_nlp, w_time, b_time, resp):
    big = pl.BlockSpec((1, B, NLP), lambda i: (i, 0, 0))
    tok = pl.BlockSpec((1, B, D), lambda i: (i, 0, 0))
    row = pl.BlockSpec((1, 1, B), lambda i: (i, 0, 0))
    return pl.pallas_call(
        _tc_body,
        grid=(L,),
        in_specs=[
            big, big, row, row,
            tok, tok, tok, tok,
            pl.BlockSpec((1, 1, D), lambda i: (i, 0, 0)),
            pl.BlockSpec((NLP, D), lambda i: (0, 0)),
            pl.BlockSpec((1, D), lambda i: (0, 0)),
            pl.BlockSpec((1, D), lambda i: (0, 0)),
            pl.BlockSpec((1, D), lambda i: (0, 0)),
            pl.BlockSpec((NR, D), lambda i: (0, 0)),
        ],
        out_specs=[tok, tok, tok],
        out_shape=[jax.ShapeDtypeStruct((L, B, D), jnp.float32)] * 3,
    )(x1_t, x2_t, el_t, r_t, g_exe, g_skill, g_oexe, g_oskill,
      pos, w_nlp, b_nlp, w_time, b_time, resp)


def kernel(input_nlp_embedding, input_exercise, input_skill, input_r,
           in_elapsed_time, output_nlp_embedding, out_exercise, out_skill,
           exercise_table, skill_table, response_table, position_table,
           W_time, b_time, W_nlp, b_nlp):
    def idx_t(a):
        return a.astype(jnp.int32).T.reshape(NW, CPW, CB)

    idx_all = jnp.stack(
        [idx_t(input_exercise), idx_t(input_skill),
         idx_t(out_exercise), idx_t(out_skill)], axis=1)

    g_exe, g_skill, g_oexe, g_oskill = _sc_gather4(
        exercise_table, skill_table, idx_all)

    def tok3d(a):
        return a.reshape(L, B, D)

    enc_t, dec_t, out_t = _tc_combine(
        input_nlp_embedding.transpose(1, 0, 2),
        output_nlp_embedding.transpose(1, 0, 2),
        in_elapsed_time[:, :, 0].T.reshape(L, 1, B),
        input_r.astype(jnp.int32).T.reshape(L, 1, B),
        tok3d(g_exe), tok3d(g_skill), tok3d(g_oexe), tok3d(g_oskill),
        position_table.reshape(L, 1, D), W_nlp, b_nlp.reshape(1, D), W_time,
        b_time.reshape(1, D), response_table)
    return (enc_t.transpose(1, 0, 2), dec_t.transpose(1, 0, 2),
            out_t.transpose(1, 0, 2))


# speedup vs baseline: 2.1745x; 1.1349x over previous
import functools

import jax
import jax.numpy as jnp
from jax import lax
from jax.experimental import pallas as pl
from jax.experimental.pallas import tpu as pltpu
from jax.experimental.pallas import tpu_sc as plsc

B, L, D = 1024, 50, 64
NLP = 768
NR = 4
BL = B * L

NC, NS = 2, 16
NW = NC * NS
CB = 80
CPW = BL // (NW * CB)
NG = 4
NBUF = 6
LAG = 2


def _sc_gather2(exe_t, skill_t, idx_all):
    mesh = plsc.VectorSubcoreMesh(core_axis_name="c", subcore_axis_name="s")

    @functools.partial(
        pl.kernel,
        mesh=mesh,
        out_type=[jax.ShapeDtypeStruct((BL, D), jnp.float32)] * 2,
        scratch_types=[
            pltpu.VMEM((NG, CPW, CB), jnp.int32),
            *[pltpu.VMEM((CB, D), jnp.float32) for _ in range(NBUF)],
            *[pltpu.SemaphoreType.DMA for _ in range(2 * NBUF)],
        ],
        compiler_params=pltpu.CompilerParams(use_tc_tiling_on_sc=False),
    )
    def k(exe_hbm, skill_hbm, idx_hbm, genc, gout, idx_v, *bufs_and_sems):
        bufs = bufs_and_sems[:NBUF]
        gsem = bufs_and_sems[NBUF:2 * NBUF]
        wsem = bufs_and_sems[2 * NBUF:]
        wid = lax.axis_index("s") * NC + lax.axis_index("c")
        base = wid * (CPW * CB)

        pltpu.sync_copy(idx_hbm.at[wid], idx_v)

        jobs = []
        for j in range(CPW):
            jobs.append((0, 1, genc, j))
            jobs.append((2, 3, gout, j))
        nj = len(jobs)
        h1 = [None] * nj
        h2 = [None] * nj
        hw = [None] * nj

        for i in range(nj + 2 * LAG):
            if i < nj:
                s = i % NBUF
                if i >= NBUF:
                    hw[i - NBUF].wait()
                ge, _, _, j = jobs[i]
                h1[i] = pltpu.async_copy(
                    exe_hbm.at[idx_v.at[ge, j]], bufs[s], gsem[s])
            if LAG <= i < nj + LAG:
                t = i - LAG
                s = t % NBUF
                _, gs, _, j = jobs[t]
                h1[t].wait()
                h2[t] = pltpu.async_copy(
                    skill_hbm.at[idx_v.at[gs, j]], bufs[s], gsem[s],
                    add=True)
            if i >= 2 * LAG:
                t = i - 2 * LAG
                s = t % NBUF
                _, _, dst, j = jobs[t]
                h2[t].wait()
                start = pl.multiple_of(base + j * CB, 8)
                hw[t] = pltpu.async_copy(
                    bufs[s], dst.at[pl.ds(start, CB)], wsem[s])
        for t in range(nj - NBUF, nj):
            hw[t].wait()

    return k(exe_t, skill_t, idx_all)


def _tc_body(x1_ref, x2_ref, el_ref, r_ref, ge_ref, go_ref,
             pos_ref, wn_ref, bn_ref, wt_ref, bt_ref, resp_ref,
             enc_ref, dec_ref, out_ref):
    w = wn_ref[...]
    pos_l = pos_ref[...].reshape(1, D)
    bn = bn_ref[...]

    x1 = x1_ref[...].reshape(B, NLP)
    y1 = jnp.dot(x1, w, preferred_element_type=jnp.float32)
    enc = y1 + bn + pos_l + ge_ref[...].reshape(B, D)
    enc_ref[...] = enc.T.reshape(1, D, B)

    el = el_ref[...].reshape(B, 1)
    r = r_ref[...].reshape(B, 1)
    onehot = (r == lax.broadcasted_iota(jnp.int32, (1, NR), 1)
              ).astype(jnp.float32)
    dec_r = jnp.dot(onehot, resp_ref[...], preferred_element_type=jnp.float32)
    dec = el * wt_ref[...] + bt_ref[...] + pos_l + dec_r
    dec_ref[...] = dec.T.reshape(1, D, B)

    x2 = x2_ref[...].reshape(B, NLP)
    y2 = jnp.dot(x2, w, preferred_element_type=jnp.float32)
    out = y2 + bn + go_ref[...].reshape(B, D)
    out_ref[...] = out.T.reshape(1, D, B)


def _tc_combine(x1_t, x2_t, el_t, r_t, g_enc, g_out,
                pos, w_nlp, b_nlp, w_time, b_time, resp):
    big = pl.BlockSpec((1, B, NLP), lambda i: (i, 0, 0))
    tok = pl.BlockSpec((1, B, D), lambda i: (i, 0, 0))
    row = pl.BlockSpec((1, 1, B), lambda i: (i, 0, 0))
    tokT = pl.BlockSpec((1, D, B), lambda i: (i, 0, 0))
    return pl.pallas_call(
        _tc_body,
        grid=(L,),
        in_specs=[
            big, big, row, row,
            tok, tok,
            pl.BlockSpec((1, 1, D), lambda i: (i, 0, 0)),
            pl.BlockSpec((NLP, D), lambda i: (0, 0)),
            pl.BlockSpec((1, D), lambda i: (0, 0)),
            pl.BlockSpec((1, D), lambda i: (0, 0)),
            pl.BlockSpec((1, D), lambda i: (0, 0)),
            pl.BlockSpec((NR, D), lambda i: (0, 0)),
        ],
        out_specs=[tokT, tokT, tokT],
        out_shape=[jax.ShapeDtypeStruct((L, D, B), jnp.float32)] * 3,
    )(x1_t, x2_t, el_t, r_t, g_enc, g_out,
      pos, w_nlp, b_nlp, w_time, b_time, resp)


def kernel(input_nlp_embedding, input_exercise, input_skill, input_r,
           in_elapsed_time, output_nlp_embedding, out_exercise, out_skill,
           exercise_table, skill_table, response_table, position_table,
           W_time, b_time, W_nlp, b_nlp):
    def idx_t(a):
        return a.astype(jnp.int32).T.reshape(NW, CPW, CB)

    idx_all = jnp.stack(
        [idx_t(input_exercise), idx_t(input_skill),
         idx_t(out_exercise), idx_t(out_skill)], axis=1)

    g_enc, g_out = _sc_gather2(exercise_table, skill_table, idx_all)

    def tok3d(a):
        return a.reshape(L, B, D)

    enc_t, dec_t, out_t = _tc_combine(
        input_nlp_embedding.transpose(1, 0, 2),
        output_nlp_embedding.transpose(1, 0, 2),
        in_elapsed_time[:, :, 0].T.reshape(L, 1, B),
        input_r.astype(jnp.int32).T.reshape(L, 1, B),
        tok3d(g_enc), tok3d(g_out),
        position_table.reshape(L, 1, D), W_nlp, b_nlp.reshape(1, D), W_time,
        b_time.reshape(1, D), response_table)
    return (enc_t.transpose(2, 0, 1), dec_t.transpose(2, 0, 1),
            out_t.transpose(2, 0, 1))
